# baseline (speedup 1.0000x reference)
import functools

import jax
import jax.numpy as jnp
from jax import lax
from jax.experimental import pallas as pl
from jax.experimental.pallas import tpu as pltpu
from jax.experimental.pallas import tpu_sc as plsc

NC = 2
NS = 16
NW = NC * NS


def _sc_mesh():
    return plsc.VectorSubcoreMesh(
        core_axis_name="c", subcore_axis_name="s", num_cores=NC, num_subcores=NS
    )


def _make_edge_scatter(NPAD, NB, D, CH, BR):
    RPT = NPAD // NS

    @functools.partial(
        pl.kernel,
        out_type=jax.ShapeDtypeStruct((NC, NPAD, D), jnp.float32),
        mesh=_sc_mesh(),
        scratch_types=[
            pltpu.VMEM((2 * BR,), jnp.int32),
            pltpu.VMEM((BR,), jnp.int32),
            pltpu.VMEM((BR, D), jnp.float32),
            pltpu.VMEM((BR, D), jnp.float32),
            pltpu.VMEM_SHARED((NPAD, D), jnp.float32),
            pltpu.SemaphoreType.DMA,
            pltpu.SemaphoreType.DMA,
        ],
    )
    def kern(z_hbm, rel_hbm, se_hbm, dst_hbm, zrow_hbm, g_out,
             sev, dv, bz, br, g_sh, sga, sgb):
        c = lax.axis_index("c")
        s = lax.axis_index("s")
        wid = c * NS + s
        pltpu.sync_copy(zrow_hbm, g_sh.at[pl.ds(s * RPT, RPT)])
        plsc.subcore_barrier()

        base = wid * NB

        def blk(i, carry):
            pltpu.sync_copy(se_hbm.at[pl.ds((base + i) * 2 * BR, 2 * BR)], sev)
            pltpu.sync_copy(dst_hbm.at[pl.ds((base + i) * BR, BR)], dv)
            cz = pltpu.async_copy(z_hbm.at[sev.at[pl.ds(0, BR)]], bz, sga)
            cr = pltpu.async_copy(rel_hbm.at[sev.at[pl.ds(BR, BR)]], br, sgb)
            cz.wait()
            pltpu.sync_copy(bz, g_sh.at[dv], add=True)
            cr.wait()
            pltpu.sync_copy(br, g_sh.at[dv], add=True)
            return carry

        lax.fori_loop(0, NB, blk, 0)
        plsc.subcore_barrier()
        pltpu.sync_copy(
            g_sh.at[pl.ds(s * RPT, RPT)], g_out.at[c, pl.ds(s * RPT, RPT)]
        )

    return kern


def _z_body(hb, pb, w1, w2, b2, out):
    out[...] = (
        jnp.dot(hb[...], w1[...], preferred_element_type=jnp.float32)
        + jnp.dot(pb[...], w2[...], preferred_element_type=jnp.float32)
        + b2[...]
    )


def _make_edge_scatter_serial(NPAD, NB, D, CH, BR):
    NCH = NB // CH
    RPT = NPAD // NS

    @functools.partial(
        pl.kernel,
        out_type=jax.ShapeDtypeStruct((NC, NPAD, D), jnp.float32),
        mesh=_sc_mesh(),
        scratch_types=[
            pltpu.VMEM((CH, BR), jnp.int32),
            pltpu.VMEM((CH, BR), jnp.int32),
            pltpu.VMEM((BR, D), jnp.float32),
            pltpu.VMEM_SHARED((NPAD, D), jnp.float32),
            pltpu.SemaphoreType.DMA,
            pltpu.SemaphoreType.DMA,
        ],
    )
    def kern(zcat_hbm, gidx_hbm, sidx_hbm, zrow_hbm, g_out,
             gix, six, rb, g_sh, sg, ss):
        c = lax.axis_index("c")
        s = lax.axis_index("s")
        wid = c * NS + s
        pltpu.sync_copy(zrow_hbm, g_sh.at[pl.ds(s * RPT, RPT)])
        plsc.subcore_barrier()

        def chunk(ci, carry):
            base = ci * CH
            pltpu.sync_copy(gidx_hbm.at[wid, pl.ds(base, CH)], gix)
            pltpu.sync_copy(sidx_hbm.at[wid, pl.ds(base, CH)], six)

            def blk(t, carry2):
                pltpu.async_copy(zcat_hbm.at[gix.at[t]], rb, sg).wait()
                pltpu.async_copy(rb, g_sh.at[six.at[t]], ss, add=True).wait()
                return carry2

            lax.fori_loop(0, CH, blk, 0)
            return carry

        lax.fori_loop(0, NCH, chunk, 0)
        plsc.subcore_barrier()
        pltpu.sync_copy(
            g_sh.at[pl.ds(s * RPT, RPT)], g_out.at[c, pl.ds(s * RPT, RPT)]
        )

    return kern


def _merge_body(g0, g1, nrm, wn, out):
    gg = g0[...] + g1[...]
    out[...] = jnp.dot(gg, wn[...], preferred_element_type=jnp.float32) * nrm[...]


def kernel(h, pos_enc, norm, prev_h, emb_rel, W_hp, b_hp, W_neighbor, edge_index, edge_type):
    N, D = h.shape
    P = pos_enc.shape[1]
    R = emb_rel.shape[0]
    E = edge_type.shape[0]
    B = 80
    CH = 16
    NPAD = 10240
    PP = 8
    EPW = NPAD
    EP = NW * EPW
    NB = EPW // B

    posp = jnp.concatenate([pos_enc, jnp.zeros((N, PP - P), jnp.float32)], axis=1)
    w1 = W_hp[:D]
    w2 = jnp.concatenate([W_hp[D:], jnp.zeros((PP - P, D), jnp.float32)], axis=0)
    b2 = b_hp.reshape(1, D)
    npad = jnp.zeros((EP - E,), jnp.int32)
    src3 = jnp.concatenate([edge_index[0], npad]).reshape(NW * NB, B)
    et3 = jnp.concatenate([edge_type, npad]).reshape(NW * NB, B)
    dstf = jnp.concatenate(
        [edge_index[1], jnp.full((EP - E,), NPAD - 1, jnp.int32)]
    )
    se = jnp.concatenate([src3, et3], axis=1).reshape(-1)
    zrow = jnp.zeros((NPAD // NS, D), jnp.float32)

    BN = 1000
    z = pl.pallas_call(
        _z_body,
        grid=(N // BN,),
        in_specs=[
            pl.BlockSpec((BN, D), lambda i: (i, 0)),
            pl.BlockSpec((BN, PP), lambda i: (i, 0)),
            pl.BlockSpec((D, D), lambda i: (0, 0)),
            pl.BlockSpec((PP, D), lambda i: (0, 0)),
            pl.BlockSpec((1, D), lambda i: (0, 0)),
        ],
        out_specs=pl.BlockSpec((BN, D), lambda i: (i, 0)),
        out_shape=jax.ShapeDtypeStruct((N, D), jnp.float32),
    )(h, posp, w1, w2, b2)

    g_parts = _make_edge_scatter(NPAD, NB, D, CH, B)(z, emb_rel, se, dstf, zrow)

    node_repr = pl.pallas_call(
        _merge_body,
        grid=(N // BN,),
        in_specs=[
            pl.BlockSpec((BN, D), lambda i: (i, 0)),
            pl.BlockSpec((BN, D), lambda i: (i, 0)),
            pl.BlockSpec((BN, 1), lambda i: (i, 0)),
            pl.BlockSpec((D, D), lambda i: (0, 0)),
        ],
        out_specs=pl.BlockSpec((BN, D), lambda i: (i, 0)),
        out_shape=jax.ShapeDtypeStruct((N, D), jnp.float32),
    )(g_parts[0], g_parts[1], norm, W_neighbor)
    return node_repr, pos_enc


# speedup vs baseline: 1.0007x; 1.0007x over previous
import functools

import jax
import jax.numpy as jnp
from jax import lax
from jax.experimental import pallas as pl
from jax.experimental.pallas import tpu as pltpu
from jax.experimental.pallas import tpu_sc as plsc

NC = 2
NS = 16
NW = NC * NS


def _sc_mesh():
    return plsc.VectorSubcoreMesh(
        core_axis_name="c", subcore_axis_name="s", num_cores=NC, num_subcores=NS
    )


def _make_edge_scatter(NPAD, NB, D, CH, BR):
    RPT = NPAD // NS

    @functools.partial(
        pl.kernel,
        out_type=jax.ShapeDtypeStruct((NC, NPAD, D), jnp.float32),
        mesh=_sc_mesh(),
        scratch_types=[
            pltpu.VMEM((2 * BR,), jnp.int32),
            pltpu.VMEM((BR,), jnp.int32),
            pltpu.VMEM((BR, D), jnp.float32),
            pltpu.VMEM((BR, D), jnp.float32),
            pltpu.VMEM_SHARED((NPAD, D), jnp.float32),
            pltpu.SemaphoreType.DMA,
            pltpu.SemaphoreType.DMA,
        ],
    )
    def kern(z_hbm, rel_hbm, se_hbm, dst_hbm, zrow_hbm, g_out,
             sev, dv, bz, br, g_sh, sga, sgb):
        c = lax.axis_index("c")
        s = lax.axis_index("s")
        wid = c * NS + s
        pltpu.sync_copy(zrow_hbm, g_sh.at[pl.ds(s * RPT, RPT)])
        plsc.subcore_barrier()

        base = wid * NB

        def blk(i, carry):
            pltpu.sync_copy(se_hbm.at[pl.ds((base + i) * 2 * BR, 2 * BR)], sev)
            pltpu.sync_copy(dst_hbm.at[pl.ds((base + i) * BR, BR)], dv)
            cz = pltpu.async_copy(z_hbm.at[sev.at[pl.ds(0, BR)]], bz, sga)
            cr = pltpu.async_copy(rel_hbm.at[sev.at[pl.ds(BR, BR)]], br, sgb)
            cz.wait()
            pltpu.sync_copy(bz, g_sh.at[dv], add=True)
            cr.wait()
            pltpu.sync_copy(br, g_sh.at[dv], add=True)
            return carry

        lax.fori_loop(0, NB, blk, 0)
        plsc.subcore_barrier()
        pltpu.sync_copy(
            g_sh.at[pl.ds(s * RPT, RPT)], g_out.at[c, pl.ds(s * RPT, RPT)]
        )

    return kern


def _z_body(hb, pb, w1, w2, b2, out):
    out[...] = (
        jnp.dot(hb[...], w1[...], preferred_element_type=jnp.float32)
        + jnp.dot(pb[...], w2[...], preferred_element_type=jnp.float32)
        + b2[...]
    )


def _make_edge_scatter_serial(NPAD, NB, D, CH, BR):
    NCH = NB // CH
    RPT = NPAD // NS

    @functools.partial(
        pl.kernel,
        out_type=jax.ShapeDtypeStruct((NC, NPAD, D), jnp.float32),
        mesh=_sc_mesh(),
        scratch_types=[
            pltpu.VMEM((CH, BR), jnp.int32),
            pltpu.VMEM((CH, BR), jnp.int32),
            pltpu.VMEM((BR, D), jnp.float32),
            pltpu.VMEM_SHARED((NPAD, D), jnp.float32),
            pltpu.SemaphoreType.DMA,
            pltpu.SemaphoreType.DMA,
        ],
    )
    def kern(zcat_hbm, gidx_hbm, sidx_hbm, zrow_hbm, g_out,
             gix, six, rb, g_sh, sg, ss):
        c = lax.axis_index("c")
        s = lax.axis_index("s")
        wid = c * NS + s
        pltpu.sync_copy(zrow_hbm, g_sh.at[pl.ds(s * RPT, RPT)])
        plsc.subcore_barrier()

        def chunk(ci, carry):
            base = ci * CH
            pltpu.sync_copy(gidx_hbm.at[wid, pl.ds(base, CH)], gix)
            pltpu.sync_copy(sidx_hbm.at[wid, pl.ds(base, CH)], six)

            def blk(t, carry2):
                pltpu.async_copy(zcat_hbm.at[gix.at[t]], rb, sg).wait()
                pltpu.async_copy(rb, g_sh.at[six.at[t]], ss, add=True).wait()
                return carry2

            lax.fori_loop(0, CH, blk, 0)
            return carry

        lax.fori_loop(0, NCH, chunk, 0)
        plsc.subcore_barrier()
        pltpu.sync_copy(
            g_sh.at[pl.ds(s * RPT, RPT)], g_out.at[c, pl.ds(s * RPT, RPT)]
        )

    return kern


def _merge_body(g0, g1, nrm, wn, out):
    gg = g0[...] + g1[...]
    out[...] = jnp.dot(gg, wn[...], preferred_element_type=jnp.float32) * nrm[...]


def kernel(h, pos_enc, norm, prev_h, emb_rel, W_hp, b_hp, W_neighbor, edge_index, edge_type):
    N, D = h.shape
    P = pos_enc.shape[1]
    R = emb_rel.shape[0]
    E = edge_type.shape[0]
    B = 80
    CH = 16
    NPAD = 10240
    PP = 8
    EPW = NPAD
    EP = NW * EPW
    NB = EPW // B

    posp = jnp.concatenate([pos_enc, jnp.zeros((N, PP - P), jnp.float32)], axis=1)
    w1 = W_hp[:D]
    w2 = jnp.concatenate([W_hp[D:], jnp.zeros((PP - P, D), jnp.float32)], axis=0)
    b2 = b_hp.reshape(1, D)
    npad = jnp.zeros((EP - E,), jnp.int32)
    src3 = jnp.concatenate([edge_index[0], npad]).reshape(NW * NB, B)
    et3 = jnp.concatenate([edge_type, npad]).reshape(NW * NB, B)
    dump = N + (jnp.arange(EP - E, dtype=jnp.int32) % (NPAD - N))
    dstf = jnp.concatenate([edge_index[1], dump])
    se = jnp.concatenate([src3, et3], axis=1).reshape(-1)
    zrow = jnp.zeros((NPAD // NS, D), jnp.float32)

    BN = 1000
    z = pl.pallas_call(
        _z_body,
        grid=(N // BN,),
        in_specs=[
            pl.BlockSpec((BN, D), lambda i: (i, 0)),
            pl.BlockSpec((BN, PP), lambda i: (i, 0)),
            pl.BlockSpec((D, D), lambda i: (0, 0)),
            pl.BlockSpec((PP, D), lambda i: (0, 0)),
            pl.BlockSpec((1, D), lambda i: (0, 0)),
        ],
        out_specs=pl.BlockSpec((BN, D), lambda i: (i, 0)),
        out_shape=jax.ShapeDtypeStruct((N, D), jnp.float32),
    )(h, posp, w1, w2, b2)

    g_parts = _make_edge_scatter(NPAD, NB, D, CH, B)(z, emb_rel, se, dstf, zrow)

    node_repr = pl.pallas_call(
        _merge_body,
        grid=(N // BN,),
        in_specs=[
            pl.BlockSpec((BN, D), lambda i: (i, 0)),
            pl.BlockSpec((BN, D), lambda i: (i, 0)),
            pl.BlockSpec((BN, 1), lambda i: (i, 0)),
            pl.BlockSpec((D, D), lambda i: (0, 0)),
        ],
        out_specs=pl.BlockSpec((BN, D), lambda i: (i, 0)),
        out_shape=jax.ShapeDtypeStruct((N, D), jnp.float32),
    )(g_parts[0], g_parts[1], norm, W_neighbor)
    return node_repr, pos_enc


# speedup vs baseline: 1.4711x; 1.4700x over previous
import functools

import jax
import jax.numpy as jnp
from jax import lax
from jax.experimental import pallas as pl
from jax.experimental.pallas import tpu as pltpu
from jax.experimental.pallas import tpu_sc as plsc

NC = 2
NS = 16
NW = NC * NS


def _sc_mesh():
    return plsc.VectorSubcoreMesh(
        core_axis_name="c", subcore_axis_name="s", num_cores=NC, num_subcores=NS
    )


def _make_edge_scatter(NPAD, NB, D, CH, BR):
    RPT = NPAD // NS

    @functools.partial(
        pl.kernel,
        out_type=jax.ShapeDtypeStruct((NC, NPAD, D), jnp.float32),
        mesh=_sc_mesh(),
        scratch_types=[
            pltpu.VMEM((BR,), jnp.int32),
            pltpu.VMEM((BR,), jnp.int32),
            pltpu.VMEM((BR,), jnp.int32),
            pltpu.VMEM((BR, D), jnp.float32),
            pltpu.VMEM((BR, D), jnp.float32),
            pltpu.VMEM_SHARED((NPAD, D), jnp.float32),
            pltpu.SemaphoreType.DMA,
            pltpu.SemaphoreType.DMA,
        ],
    )
    def kern(z_hbm, rel_hbm, src_hbm, dst_hbm, et_hbm, zrow_hbm, g_out,
             sv, dv, ev, bz, br, g_sh, sga, sgb):
        c = lax.axis_index("c")
        s = lax.axis_index("s")
        wid = c * NS + s
        pltpu.sync_copy(zrow_hbm, g_sh.at[pl.ds(s * RPT, RPT)])
        plsc.subcore_barrier()

        base = wid * NB * BR

        def blk(i, carry):
            off = base + i * BR
            pltpu.sync_copy(src_hbm.at[pl.ds(off, BR)], sv)
            pltpu.sync_copy(et_hbm.at[pl.ds(off, BR)], ev)
            pltpu.sync_copy(dst_hbm.at[pl.ds(off, BR)], dv)
            cz = pltpu.async_copy(z_hbm.at[sv], bz, sga)
            cr = pltpu.async_copy(rel_hbm.at[ev], br, sgb)
            cz.wait()
            pltpu.sync_copy(bz, g_sh.at[dv], add=True)
            cr.wait()
            pltpu.sync_copy(br, g_sh.at[dv], add=True)
            return carry

        lax.fori_loop(0, NB, blk, 0)
        plsc.subcore_barrier()
        pltpu.sync_copy(
            g_sh.at[pl.ds(s * RPT, RPT)], g_out.at[c, pl.ds(s * RPT, RPT)]
        )

    return kern


def _z_body(hb, pb, w1, w2, b2, out):
    out[...] = (
        jnp.dot(hb[...], w1[...], preferred_element_type=jnp.float32)
        + jnp.dot(pb[...], w2[...], preferred_element_type=jnp.float32)
        + b2[...]
    )


def _make_edge_scatter_serial(NPAD, NB, D, CH, BR):
    NCH = NB // CH
    RPT = NPAD // NS

    @functools.partial(
        pl.kernel,
        out_type=jax.ShapeDtypeStruct((NC, NPAD, D), jnp.float32),
        mesh=_sc_mesh(),
        scratch_types=[
            pltpu.VMEM((CH, BR), jnp.int32),
            pltpu.VMEM((CH, BR), jnp.int32),
            pltpu.VMEM((BR, D), jnp.float32),
            pltpu.VMEM_SHARED((NPAD, D), jnp.float32),
            pltpu.SemaphoreType.DMA,
            pltpu.SemaphoreType.DMA,
        ],
    )
    def kern(zcat_hbm, gidx_hbm, sidx_hbm, zrow_hbm, g_out,
             gix, six, rb, g_sh, sg, ss):
        c = lax.axis_index("c")
        s = lax.axis_index("s")
        wid = c * NS + s
        pltpu.sync_copy(zrow_hbm, g_sh.at[pl.ds(s * RPT, RPT)])
        plsc.subcore_barrier()

        def chunk(ci, carry):
            base = ci * CH
            pltpu.sync_copy(gidx_hbm.at[wid, pl.ds(base, CH)], gix)
            pltpu.sync_copy(sidx_hbm.at[wid, pl.ds(base, CH)], six)

            def blk(t, carry2):
                pltpu.async_copy(zcat_hbm.at[gix.at[t]], rb, sg).wait()
                pltpu.async_copy(rb, g_sh.at[six.at[t]], ss, add=True).wait()
                return carry2

            lax.fori_loop(0, CH, blk, 0)
            return carry

        lax.fori_loop(0, NCH, chunk, 0)
        plsc.subcore_barrier()
        pltpu.sync_copy(
            g_sh.at[pl.ds(s * RPT, RPT)], g_out.at[c, pl.ds(s * RPT, RPT)]
        )

    return kern


def _merge_body(g0, g1, nrm, wn, out):
    gg = g0[...] + g1[...]
    out[...] = jnp.dot(gg, wn[...], preferred_element_type=jnp.float32) * nrm[...]


def kernel(h, pos_enc, norm, prev_h, emb_rel, W_hp, b_hp, W_neighbor, edge_index, edge_type):
    N, D = h.shape
    P = pos_enc.shape[1]
    R = emb_rel.shape[0]
    E = edge_type.shape[0]
    B = 80
    CH = 16
    NPAD = 10240
    PP = 8
    EPW = E // NW
    NB = EPW // B

    posp = jnp.concatenate([pos_enc, jnp.zeros((N, PP - P), jnp.float32)], axis=1)
    w1 = W_hp[:D]
    w2 = jnp.concatenate([W_hp[D:], jnp.zeros((PP - P, D), jnp.float32)], axis=0)
    b2 = b_hp.reshape(1, D)
    src = edge_index[0]
    dst = edge_index[1]
    zrow = jnp.zeros((NPAD // NS, D), jnp.float32)

    BN = 1000
    z = pl.pallas_call(
        _z_body,
        grid=(N // BN,),
        in_specs=[
            pl.BlockSpec((BN, D), lambda i: (i, 0)),
            pl.BlockSpec((BN, PP), lambda i: (i, 0)),
            pl.BlockSpec((D, D), lambda i: (0, 0)),
            pl.BlockSpec((PP, D), lambda i: (0, 0)),
            pl.BlockSpec((1, D), lambda i: (0, 0)),
        ],
        out_specs=pl.BlockSpec((BN, D), lambda i: (i, 0)),
        out_shape=jax.ShapeDtypeStruct((N, D), jnp.float32),
    )(h, posp, w1, w2, b2)

    g_parts = _make_edge_scatter(NPAD, NB, D, CH, B)(z, emb_rel, src, dst, edge_type, zrow)

    node_repr = pl.pallas_call(
        _merge_body,
        grid=(N // BN,),
        in_specs=[
            pl.BlockSpec((BN, D), lambda i: (i, 0)),
            pl.BlockSpec((BN, D), lambda i: (i, 0)),
            pl.BlockSpec((BN, 1), lambda i: (i, 0)),
            pl.BlockSpec((D, D), lambda i: (0, 0)),
        ],
        out_specs=pl.BlockSpec((BN, D), lambda i: (i, 0)),
        out_shape=jax.ShapeDtypeStruct((N, D), jnp.float32),
    )(g_parts[0], g_parts[1], norm, W_neighbor)
    return node_repr, pos_enc
